# baseline (speedup 1.0000x reference)
import functools

import jax
import jax.numpy as jnp
from jax import lax
from jax.experimental import pallas as pl
from jax.experimental.pallas import tpu as pltpu
from jax.experimental.pallas import tpu_sc as plsc

_INFO = plsc.get_sparse_core_info()
_NC = _INFO.num_cores
_NS = _INFO.num_subcores
_NW = _NC * _NS
_L = 16


def _mesh():
    return plsc.VectorSubcoreMesh(core_axis_name="c", subcore_axis_name="s")


def _wid():
    return lax.axis_index("s") * _NC + lax.axis_index("c")


def _splat(val):
    return jnp.full((_L,), val, jnp.int32)


@functools.lru_cache(maxsize=None)
def _build_transpose(vocab, dim):
    n_full = vocab // 128
    tail = vocab - n_full * 128
    base_t = n_full // _NW
    extra = n_full - base_t * _NW
    max_t = base_t + (1 if extra else 0)
    n_pairs = (max_t + 1) // 2

    in_args = 2 if tail else 1

    @functools.partial(
        pl.kernel,
        mesh=_mesh(),
        out_type=jax.ShapeDtypeStruct((vocab, 128), jnp.float32),
        scratch_types=[
            pltpu.VMEM((2, dim, 128), jnp.float32),
            pltpu.VMEM((2, 128, 128), jnp.float32),
            pltpu.SemaphoreType.DMA((2,)),
            pltpu.SemaphoreType.DMA((2,)),
        ],
        compiler_params=pltpu.CompilerParams(use_tc_tiling_on_sc=True, needs_layout_passes=False),
    )
    def transpose_kernel(*refs):
        if tail:
            tt_hbm, t_tail_hbm, t2_hbm, slabs, tbufs, gsems, wsems = refs
        else:
            tt_hbm, t2_hbm, slabs, tbufs, gsems, wsems = refs
            t_tail_hbm = None
        w = _wid()
        count = base_t + jnp.where(w < extra, 1, 0)

        def c0_of(k):
            return (k * _NW + w) * 128

        def start_gather(k, b):
            pltpu.make_async_copy(
                tt_hbm.at[:, pl.ds(c0_of(k), 128)], slabs.at[b], gsems.at[b]
            ).start()

        def wait_gather(k, b):
            pltpu.make_async_copy(
                tt_hbm.at[:, pl.ds(c0_of(k), 128)], slabs.at[b], gsems.at[b]
            ).wait()

        def transpose_slab(b):
            def body(vb, carry):
                for vi in range(4):
                    v = vb * 4 + vi
                    for kk in range(dim // _L):
                        vals = plsc.load_gather(
                            slabs.at[b],
                            [lax.iota(jnp.int32, _L) + kk * _L, _splat(v)],
                        )
                        tbufs.at[b][v, pl.ds(kk * _L, _L)] = vals
                return carry

            lax.fori_loop(0, 32, body, 0)

        def start_write(k, b):
            pltpu.make_async_copy(
                tbufs.at[b], t2_hbm.at[pl.ds(c0_of(k), 128)], wsems.at[b]
            ).start()

        def wait_write(k, b):
            pltpu.make_async_copy(
                tbufs.at[b], t2_hbm.at[pl.ds(c0_of(k), 128)], wsems.at[b]
            ).wait()

        start_gather(0, 0)
        start_gather(1, 1)

        def pair(g, carry):
            for b in range(2):
                k = g * 2 + b

                @pl.when(k < count)
                def _():
                    wait_gather(k, b)

                    @pl.when(k >= 2)
                    def _():
                        wait_write(k - 2, b)

                    transpose_slab(b)
                    start_write(k, b)

                    @pl.when(k + 2 < count)
                    def _():
                        start_gather(k + 2, b)

            return carry

        lax.fori_loop(0, n_pairs, pair, 0)
        wait_write(0, 0)
        wait_write(0, 1)

        if tail:
            @pl.when(w == _NW - 1)
            def _():
                c0 = n_full * 128
                pltpu.sync_copy(t_tail_hbm,
                                tbufs.at[0].at[pl.ds(0, tail)])
                pltpu.sync_copy(tbufs.at[0].at[pl.ds(0, tail)],
                                t2_hbm.at[pl.ds(c0, tail)])

    return transpose_kernel


@functools.lru_cache(maxsize=None)
def _build_gather(bsz, hist, vocab, dim):
    nbuf = 4
    assert bsz == _NW * 128 and hist % nbuf == 0 and dim % 8 == 0
    dr = dim // 8
    n_groups = hist // nbuf

    @functools.partial(
        pl.kernel,
        mesh=_mesh(),
        out_type=jax.ShapeDtypeStruct((hist, dr, bsz // 128, 8, 128),
                                      jnp.float32),
        scratch_types=[
            pltpu.VMEM((hist, 128), jnp.int32),
            pltpu.VMEM((nbuf, 128, 128), jnp.float32),
            pltpu.VMEM((nbuf, dr, 8, 128), jnp.float32),
            pltpu.SemaphoreType.DMA((nbuf,)),
            pltpu.SemaphoreType.DMA((nbuf,)),
        ],
        compiler_params=pltpu.CompilerParams(use_tc_tiling_on_sc=True, needs_layout_passes=False),
    )
    def gather_kernel(xt_hbm, t2_hbm, out_hbm, idx_v, gbufs, tbufs,
                      gsems, wsems):
        w = _wid()
        pltpu.sync_copy(xt_hbm.at[:, pl.ds(w * 128, 128)], idx_v)

        def start_gather(h, b):
            pltpu.make_async_copy(
                t2_hbm.at[idx_v.at[h]], gbufs.at[b], gsems.at[b]
            ).start()

        def wait_gather(h, b):
            pltpu.make_async_copy(
                t2_hbm.at[idx_v.at[h]], gbufs.at[b], gsems.at[b]
            ).wait()

        def transpose_block(b):
            def body(rr, carry):
                for r in range(8):
                    for j in range(8):
                        vals = plsc.load_gather(
                            gbufs.at[b],
                            [lax.iota(jnp.int32, _L) + j * _L,
                             _splat(rr * 8 + r)],
                        )
                        tbufs.at[b][rr, r, pl.ds(j * _L, _L)] = vals
                return carry

            lax.fori_loop(0, dr, body, 0)

        def start_write(h, b):
            pltpu.make_async_copy(
                tbufs.at[b], out_hbm.at[h].at[:, w], wsems.at[b]
            ).start()

        def wait_write(h, b):
            pltpu.make_async_copy(
                tbufs.at[b], out_hbm.at[h].at[:, w], wsems.at[b]
            ).wait()

        for b in range(nbuf):
            start_gather(b, b)

        def group(g, carry):
            for b in range(nbuf):
                h = g * nbuf + b
                wait_gather(h, b)

                @pl.when(h >= nbuf)
                def _():
                    wait_write(h - nbuf, b)

                transpose_block(b)
                start_write(h, b)

                @pl.when(h + nbuf < hist)
                def _():
                    start_gather(h + nbuf, b)

            return carry

        lax.fori_loop(0, n_groups, group, 0)
        for b in range(nbuf):
            wait_write(0, b)

    return gather_kernel


def kernel(x, table):
    bsz, hist = x.shape
    vocab, dim = table.shape
    tt = table.T
    xt = x.astype(jnp.int32).T
    n_full = vocab // 128
    tail = vocab - n_full * 128
    if tail:
        t_tailp = jnp.pad(table[n_full * 128:], ((0, 0), (0, 128 - dim)))
        t2 = _build_transpose(vocab, dim)(tt, t_tailp)
    else:
        t2 = _build_transpose(vocab, dim)(tt)
    outp = _build_gather(bsz, hist, vocab, dim)(xt, t2)
    return outp.transpose(2, 4, 0, 1, 3).reshape(bsz, hist, dim)


# speedup vs baseline: 1.2356x; 1.2356x over previous
import functools

import jax
import jax.numpy as jnp
from jax import lax
from jax.experimental import pallas as pl
from jax.experimental.pallas import tpu as pltpu
from jax.experimental.pallas import tpu_sc as plsc

_INFO = plsc.get_sparse_core_info()
_NC = _INFO.num_cores
_NS = _INFO.num_subcores
_NW = _NC * _NS
_L = 16
_PAD = 136


def _mesh():
    return plsc.VectorSubcoreMesh(core_axis_name="c", subcore_axis_name="s")


def _wid():
    return lax.axis_index("s") * _NC + lax.axis_index("c")


def _splat(val):
    return jnp.full((_L,), val, jnp.int32)


@functools.lru_cache(maxsize=None)
def _build_transpose(vocab, dim):
    n_full = vocab // 128
    tail = vocab - n_full * 128
    base_t = n_full // _NW
    extra = n_full - base_t * _NW
    max_t = base_t + (1 if extra else 0)
    n_pairs = (max_t + 1) // 2

    @functools.partial(
        pl.kernel,
        mesh=_mesh(),
        out_type=jax.ShapeDtypeStruct((vocab, 128), jnp.float32),
        scratch_types=[
            pltpu.VMEM((2, dim, 128), jnp.float32),
            pltpu.VMEM((2, 128, _PAD), jnp.float32),
            pltpu.SemaphoreType.DMA((2,)),
            pltpu.SemaphoreType.DMA((2,)),
        ],
        compiler_params=pltpu.CompilerParams(use_tc_tiling_on_sc=True, needs_layout_passes=False),
    )
    def transpose_kernel(*refs):
        if tail:
            tt_hbm, t_tail_hbm, t2_hbm, slabs, tbufs, gsems, wsems = refs
        else:
            tt_hbm, t2_hbm, slabs, tbufs, gsems, wsems = refs
            t_tail_hbm = None
        w = _wid()
        count = base_t + jnp.where(w < extra, 1, 0)
        iotas = [lax.iota(jnp.int32, _L) + kk * _L for kk in range(8)]

        def c0_of(k):
            return (k * _NW + w) * 128

        def start_gather(k, b):
            pltpu.make_async_copy(
                tt_hbm.at[:, pl.ds(c0_of(k), 128)], slabs.at[b], gsems.at[b]
            ).start()

        def wait_gather(k, b):
            pltpu.make_async_copy(
                tt_hbm.at[:, pl.ds(c0_of(k), 128)], slabs.at[b], gsems.at[b]
            ).wait()

        def transpose_slab(b):
            def body(dq, carry):
                for di in range(4):
                    d = dq * 4 + di
                    col = _splat(d)
                    for kk in range(8):
                        vals = slabs.at[b][d, pl.ds(kk * _L, _L)]
                        plsc.store_scatter(tbufs.at[b], [iotas[kk], col],
                                           vals)
                return carry

            lax.fori_loop(0, dim // 4, body, 0)

        def start_write(k, b):
            pltpu.make_async_copy(
                tbufs.at[b].at[:, pl.ds(0, 128)],
                t2_hbm.at[pl.ds(c0_of(k), 128)],
                wsems.at[b],
            ).start()

        def wait_write(k, b):
            pltpu.make_async_copy(
                tbufs.at[b].at[:, pl.ds(0, 128)],
                t2_hbm.at[pl.ds(c0_of(k), 128)],
                wsems.at[b],
            ).wait()

        start_gather(0, 0)
        start_gather(1, 1)

        def pair(g, carry):
            for b in range(2):
                k = g * 2 + b

                @pl.when(k < count)
                def _():
                    wait_gather(k, b)

                    @pl.when(k >= 2)
                    def _():
                        wait_write(k - 2, b)

                    transpose_slab(b)
                    start_write(k, b)

                    @pl.when(k + 2 < count)
                    def _():
                        start_gather(k + 2, b)

            return carry

        lax.fori_loop(0, n_pairs, pair, 0)
        wait_write(0, 0)
        wait_write(0, 1)

        if tail:
            @pl.when(w == _NW - 1)
            def _():
                c0 = n_full * 128
                pltpu.sync_copy(t_tail_hbm, slabs.at[0].at[pl.ds(0, tail)])
                pltpu.sync_copy(slabs.at[0].at[pl.ds(0, tail)],
                                t2_hbm.at[pl.ds(c0, tail)])

    return transpose_kernel


@functools.lru_cache(maxsize=None)
def _build_gather(bsz, hist, vocab, dim):
    nbuf = 2
    assert bsz == _NW * 128 and hist % nbuf == 0 and dim % _L == 0
    dr = dim // 8
    n_groups = hist // nbuf

    @functools.partial(
        pl.kernel,
        mesh=_mesh(),
        out_type=jax.ShapeDtypeStruct((hist, dr, bsz // 128, 8, 128),
                                      jnp.float32),
        scratch_types=[
            pltpu.VMEM((hist, 128), jnp.int32),
            pltpu.VMEM((nbuf, 128, 128), jnp.float32),
            pltpu.VMEM((nbuf, dr, 8, _PAD), jnp.float32),
            pltpu.SemaphoreType.DMA((nbuf,)),
            pltpu.SemaphoreType.DMA((nbuf,)),
        ],
        compiler_params=pltpu.CompilerParams(use_tc_tiling_on_sc=True, needs_layout_passes=False),
    )
    def gather_kernel(xt_hbm, t2_hbm, out_hbm, idx_v, gbufs, tbufs,
                      gsems, wsems):
        w = _wid()
        pltpu.sync_copy(xt_hbm.at[:, pl.ds(w * 128, 128)], idx_v)
        iotas = [lax.iota(jnp.int32, _L) + kk * _L
                 for kk in range(dim // _L)]
        r_hi = [i // 8 for i in iotas]
        r_lo = [i % 8 for i in iotas]

        def start_gather(h, b):
            pltpu.make_async_copy(
                t2_hbm.at[idx_v.at[h]], gbufs.at[b], gsems.at[b]
            ).start()

        def wait_gather(h, b):
            pltpu.make_async_copy(
                t2_hbm.at[idx_v.at[h]], gbufs.at[b], gsems.at[b]
            ).wait()

        def transpose_block(b):
            def body(cq, carry):
                for ci in range(4):
                    c = cq * 4 + ci
                    col = _splat(c)
                    for kk in range(dim // _L):
                        vals = gbufs.at[b][c, pl.ds(kk * _L, _L)]
                        plsc.store_scatter(tbufs.at[b],
                                           [r_hi[kk], r_lo[kk], col], vals)
                return carry

            lax.fori_loop(0, 32, body, 0)

        def start_write(h, b):
            pltpu.make_async_copy(
                tbufs.at[b].at[:, :, pl.ds(0, 128)],
                out_hbm.at[h].at[:, w],
                wsems.at[b],
            ).start()

        def wait_write(h, b):
            pltpu.make_async_copy(
                tbufs.at[b].at[:, :, pl.ds(0, 128)],
                out_hbm.at[h].at[:, w],
                wsems.at[b],
            ).wait()

        for b in range(nbuf):
            start_gather(b, b)

        def group(g, carry):
            for b in range(nbuf):
                h = g * nbuf + b
                wait_gather(h, b)

                @pl.when(h >= nbuf)
                def _():
                    wait_write(h - nbuf, b)

                transpose_block(b)
                start_write(h, b)

                @pl.when(h + nbuf < hist)
                def _():
                    start_gather(h + nbuf, b)

            return carry

        lax.fori_loop(0, n_groups, group, 0)
        for b in range(nbuf):
            wait_write(0, b)

    return gather_kernel


def kernel(x, table):
    bsz, hist = x.shape
    vocab, dim = table.shape
    tt = table.T
    xt = x.astype(jnp.int32).T
    n_full = vocab // 128
    tail = vocab - n_full * 128
    if tail:
        t_tailp = jnp.pad(table[n_full * 128:], ((0, 0), (0, 128 - dim)))
        t2 = _build_transpose(vocab, dim)(tt, t_tailp)
    else:
        t2 = _build_transpose(vocab, dim)(tt)
    outp = _build_gather(bsz, hist, vocab, dim)(xt, t2)
    return outp.transpose(2, 4, 0, 1, 3).reshape(bsz, hist, dim)


# speedup vs baseline: 1.7183x; 1.3906x over previous
import functools

import jax
import jax.numpy as jnp
from jax import lax
from jax.experimental import pallas as pl
from jax.experimental.pallas import tpu as pltpu
from jax.experimental.pallas import tpu_sc as plsc

_INFO = plsc.get_sparse_core_info()
_NC = _INFO.num_cores
_NS = _INFO.num_subcores
_NW = _NC * _NS
_L = 16
_PAD = 136


def _mesh():
    return plsc.VectorSubcoreMesh(core_axis_name="c", subcore_axis_name="s")


def _wid():
    return lax.axis_index("s") * _NC + lax.axis_index("c")


def _splat(val):
    return jnp.full((_L,), val, jnp.int32)


@functools.lru_cache(maxsize=None)
def _build_transpose(vocab, dim):
    n_full = vocab // 128
    tail = vocab - n_full * 128
    base_t = n_full // _NW
    extra = n_full - base_t * _NW
    max_t = base_t + (1 if extra else 0)
    n_pairs = (max_t + 1) // 2

    @functools.partial(
        pl.kernel,
        mesh=_mesh(),
        out_type=jax.ShapeDtypeStruct((vocab, 128), jnp.float32),
        scratch_types=[
            pltpu.VMEM((2, dim, 128), jnp.float32),
            pltpu.VMEM((2, 128, _PAD), jnp.float32),
            pltpu.SemaphoreType.DMA((2,)),
            pltpu.SemaphoreType.DMA((2,)),
        ],
        compiler_params=pltpu.CompilerParams(use_tc_tiling_on_sc=True, needs_layout_passes=False),
    )
    def transpose_kernel(*refs):
        if tail:
            tt_hbm, t_tail_hbm, t2_hbm, slabs, tbufs, gsems, wsems = refs
        else:
            tt_hbm, t2_hbm, slabs, tbufs, gsems, wsems = refs
            t_tail_hbm = None
        w = _wid()
        count = base_t + jnp.where(w < extra, 1, 0)
        iotas = [lax.iota(jnp.int32, _L) + kk * _L for kk in range(8)]

        def c0_of(k):
            return (k * _NW + w) * 128

        def start_gather(k, b):
            pltpu.make_async_copy(
                tt_hbm.at[:, pl.ds(c0_of(k), 128)], slabs.at[b], gsems.at[b]
            ).start()

        def wait_gather(k, b):
            pltpu.make_async_copy(
                tt_hbm.at[:, pl.ds(c0_of(k), 128)], slabs.at[b], gsems.at[b]
            ).wait()

        def transpose_slab(b):
            @plsc.parallel_loop(0, dim, step=4, unroll=2)
            def _(d0):
                for di in range(4):
                    d = d0 + di
                    col = _splat(d)
                    for kk in range(8):
                        vals = slabs.at[b][d, pl.ds(kk * _L, _L)]
                        plsc.store_scatter(tbufs.at[b], [iotas[kk], col],
                                           vals)

        def start_write(k, b):
            pltpu.make_async_copy(
                tbufs.at[b].at[:, pl.ds(0, 128)],
                t2_hbm.at[pl.ds(c0_of(k), 128)],
                wsems.at[b],
            ).start()

        def wait_write(k, b):
            pltpu.make_async_copy(
                tbufs.at[b].at[:, pl.ds(0, 128)],
                t2_hbm.at[pl.ds(c0_of(k), 128)],
                wsems.at[b],
            ).wait()

        start_gather(0, 0)
        start_gather(1, 1)

        def pair(g, carry):
            for b in range(2):
                k = g * 2 + b

                @pl.when(k < count)
                def _():
                    wait_gather(k, b)

                    @pl.when(k >= 2)
                    def _():
                        wait_write(k - 2, b)

                    transpose_slab(b)
                    start_write(k, b)

                    @pl.when(k + 2 < count)
                    def _():
                        start_gather(k + 2, b)

            return carry

        lax.fori_loop(0, n_pairs, pair, 0)
        wait_write(0, 0)
        wait_write(0, 1)

        if tail:
            @pl.when(w == _NW - 1)
            def _():
                c0 = n_full * 128
                pltpu.sync_copy(t_tail_hbm, slabs.at[0].at[pl.ds(0, tail)])
                pltpu.sync_copy(slabs.at[0].at[pl.ds(0, tail)],
                                t2_hbm.at[pl.ds(c0, tail)])

    return transpose_kernel


@functools.lru_cache(maxsize=None)
def _build_gather(bsz, hist, vocab, dim):
    nbuf = 2
    assert bsz == _NW * 128 and hist % nbuf == 0 and dim % _L == 0
    dr = dim // 8
    n_groups = hist // nbuf

    @functools.partial(
        pl.kernel,
        mesh=_mesh(),
        out_type=jax.ShapeDtypeStruct((hist, dr, bsz // 128, 8, 128),
                                      jnp.float32),
        scratch_types=[
            pltpu.VMEM((hist, 128), jnp.int32),
            pltpu.VMEM((nbuf, 128, 128), jnp.float32),
            pltpu.VMEM((nbuf, dr, 8, _PAD), jnp.float32),
            pltpu.SemaphoreType.DMA((nbuf,)),
            pltpu.SemaphoreType.DMA((nbuf,)),
        ],
        compiler_params=pltpu.CompilerParams(use_tc_tiling_on_sc=True, needs_layout_passes=False),
    )
    def gather_kernel(xt_hbm, t2_hbm, out_hbm, idx_v, gbufs, tbufs,
                      gsems, wsems):
        w = _wid()
        pltpu.sync_copy(xt_hbm.at[:, pl.ds(w * 128, 128)], idx_v)
        iotas = [lax.iota(jnp.int32, _L) + kk * _L
                 for kk in range(dim // _L)]
        r_hi = [i // 8 for i in iotas]
        r_lo = [i % 8 for i in iotas]

        def start_gather(h, b):
            pltpu.make_async_copy(
                t2_hbm.at[idx_v.at[h]], gbufs.at[b], gsems.at[b]
            ).start()

        def wait_gather(h, b):
            pltpu.make_async_copy(
                t2_hbm.at[idx_v.at[h]], gbufs.at[b], gsems.at[b]
            ).wait()

        def transpose_block(b):
            @plsc.parallel_loop(0, 128, step=4, unroll=2)
            def _(c0):
                for ci in range(4):
                    c = c0 + ci
                    col = _splat(c)
                    for kk in range(dim // _L):
                        vals = gbufs.at[b][c, pl.ds(kk * _L, _L)]
                        plsc.store_scatter(tbufs.at[b],
                                           [r_hi[kk], r_lo[kk], col], vals)

        def start_write(h, b):
            pltpu.make_async_copy(
                tbufs.at[b].at[:, :, pl.ds(0, 128)],
                out_hbm.at[h].at[:, w],
                wsems.at[b],
            ).start()

        def wait_write(h, b):
            pltpu.make_async_copy(
                tbufs.at[b].at[:, :, pl.ds(0, 128)],
                out_hbm.at[h].at[:, w],
                wsems.at[b],
            ).wait()

        for b in range(nbuf):
            start_gather(b, b)

        def group(g, carry):
            for b in range(nbuf):
                h = g * nbuf + b
                wait_gather(h, b)

                @pl.when(h >= nbuf)
                def _():
                    wait_write(h - nbuf, b)

                transpose_block(b)
                start_write(h, b)

                @pl.when(h + nbuf < hist)
                def _():
                    start_gather(h + nbuf, b)

            return carry

        lax.fori_loop(0, n_groups, group, 0)
        for b in range(nbuf):
            wait_write(0, b)

    return gather_kernel


def kernel(x, table):
    bsz, hist = x.shape
    vocab, dim = table.shape
    tt = table.T
    xt = x.astype(jnp.int32).T
    n_full = vocab // 128
    tail = vocab - n_full * 128
    if tail:
        t_tailp = jnp.pad(table[n_full * 128:], ((0, 0), (0, 128 - dim)))
        t2 = _build_transpose(vocab, dim)(tt, t_tailp)
    else:
        t2 = _build_transpose(vocab, dim)(tt)
    outp = _build_gather(bsz, hist, vocab, dim)(xt, t2)
    return outp.transpose(2, 4, 0, 1, 3).reshape(bsz, hist, dim)


# speedup vs baseline: 1.7230x; 1.0028x over previous
import functools

import jax
import jax.numpy as jnp
from jax import lax
from jax.experimental import pallas as pl
from jax.experimental.pallas import tpu as pltpu
from jax.experimental.pallas import tpu_sc as plsc

_INFO = plsc.get_sparse_core_info()
_NC = _INFO.num_cores
_NS = _INFO.num_subcores
_NW = _NC * _NS
_L = 16
_PAD = 136


def _mesh():
    return plsc.VectorSubcoreMesh(core_axis_name="c", subcore_axis_name="s")


def _wid():
    return lax.axis_index("s") * _NC + lax.axis_index("c")


def _splat(val):
    return jnp.full((_L,), val, jnp.int32)


@functools.lru_cache(maxsize=None)
def _build_transpose(vocab, dim):
    n_full = vocab // 128
    tail = vocab - n_full * 128
    base_t = n_full // _NW
    extra = n_full - base_t * _NW
    max_t = base_t + (1 if extra else 0)
    n_pairs = (max_t + 1) // 2

    @functools.partial(
        pl.kernel,
        mesh=_mesh(),
        out_type=jax.ShapeDtypeStruct((vocab, 128), jnp.float32),
        scratch_types=[
            pltpu.VMEM((2, dim, 128), jnp.float32),
            pltpu.VMEM((2, 128, _PAD), jnp.float32),
            pltpu.SemaphoreType.DMA((2,)),
            pltpu.SemaphoreType.DMA((2,)),
        ],
        compiler_params=pltpu.CompilerParams(use_tc_tiling_on_sc=True, needs_layout_passes=False),
    )
    def transpose_kernel(*refs):
        if tail:
            tt_hbm, t_tail_hbm, t2_hbm, slabs, tbufs, gsems, wsems = refs
        else:
            tt_hbm, t2_hbm, slabs, tbufs, gsems, wsems = refs
            t_tail_hbm = None
        w = _wid()
        count = base_t + jnp.where(w < extra, 1, 0)
        iotas = [lax.iota(jnp.int32, _L) + kk * _L for kk in range(8)]

        def c0_of(k):
            return (k * _NW + w) * 128

        def start_gather(k, b):
            pltpu.make_async_copy(
                tt_hbm.at[:, pl.ds(c0_of(k), 128)], slabs.at[b], gsems.at[b]
            ).start()

        def wait_gather(k, b):
            pltpu.make_async_copy(
                tt_hbm.at[:, pl.ds(c0_of(k), 128)], slabs.at[b], gsems.at[b]
            ).wait()

        def transpose_slab(b):
            @plsc.parallel_loop(0, dim, step=4, unroll=4)
            def _(d0):
                for di in range(4):
                    d = d0 + di
                    col = _splat(d)
                    for kk in range(8):
                        vals = slabs.at[b][d, pl.ds(kk * _L, _L)]
                        plsc.store_scatter(tbufs.at[b], [iotas[kk], col],
                                           vals)

        def start_write(k, b):
            pltpu.make_async_copy(
                tbufs.at[b].at[:, pl.ds(0, 128)],
                t2_hbm.at[pl.ds(c0_of(k), 128)],
                wsems.at[b],
            ).start()

        def wait_write(k, b):
            pltpu.make_async_copy(
                tbufs.at[b].at[:, pl.ds(0, 128)],
                t2_hbm.at[pl.ds(c0_of(k), 128)],
                wsems.at[b],
            ).wait()

        start_gather(0, 0)
        start_gather(1, 1)

        def pair(g, carry):
            for b in range(2):
                k = g * 2 + b

                @pl.when(k < count)
                def _():
                    wait_gather(k, b)

                    @pl.when(k >= 2)
                    def _():
                        wait_write(k - 2, b)

                    transpose_slab(b)
                    start_write(k, b)

                    @pl.when(k + 2 < count)
                    def _():
                        start_gather(k + 2, b)

            return carry

        lax.fori_loop(0, n_pairs, pair, 0)
        wait_write(0, 0)
        wait_write(0, 1)

        if tail:
            @pl.when(w == _NW - 1)
            def _():
                c0 = n_full * 128
                pltpu.sync_copy(t_tail_hbm, slabs.at[0].at[pl.ds(0, tail)])
                pltpu.sync_copy(slabs.at[0].at[pl.ds(0, tail)],
                                t2_hbm.at[pl.ds(c0, tail)])

    return transpose_kernel


@functools.lru_cache(maxsize=None)
def _build_gather(bsz, hist, vocab, dim):
    nbuf = 2
    assert bsz == _NW * 128 and hist % nbuf == 0 and dim % _L == 0
    dr = dim // 8
    n_groups = hist // nbuf

    @functools.partial(
        pl.kernel,
        mesh=_mesh(),
        out_type=jax.ShapeDtypeStruct((hist, dr, bsz // 128, 8, 128),
                                      jnp.float32),
        scratch_types=[
            pltpu.VMEM((hist, 128), jnp.int32),
            pltpu.VMEM((nbuf, 128, 128), jnp.float32),
            pltpu.VMEM((nbuf, dr, 8, _PAD), jnp.float32),
            pltpu.SemaphoreType.DMA((nbuf,)),
            pltpu.SemaphoreType.DMA((nbuf,)),
        ],
        compiler_params=pltpu.CompilerParams(use_tc_tiling_on_sc=True, needs_layout_passes=False),
    )
    def gather_kernel(xt_hbm, t2_hbm, out_hbm, idx_v, gbufs, tbufs,
                      gsems, wsems):
        w = _wid()
        pltpu.sync_copy(xt_hbm.at[:, pl.ds(w * 128, 128)], idx_v)
        iotas = [lax.iota(jnp.int32, _L) + kk * _L
                 for kk in range(dim // _L)]
        r_hi = [i // 8 for i in iotas]
        r_lo = [i % 8 for i in iotas]

        def start_gather(h, b):
            pltpu.make_async_copy(
                t2_hbm.at[idx_v.at[h]], gbufs.at[b], gsems.at[b]
            ).start()

        def wait_gather(h, b):
            pltpu.make_async_copy(
                t2_hbm.at[idx_v.at[h]], gbufs.at[b], gsems.at[b]
            ).wait()

        def transpose_block(b):
            @plsc.parallel_loop(0, 128, step=4, unroll=4)
            def _(c0):
                for ci in range(4):
                    c = c0 + ci
                    col = _splat(c)
                    for kk in range(dim // _L):
                        vals = gbufs.at[b][c, pl.ds(kk * _L, _L)]
                        plsc.store_scatter(tbufs.at[b],
                                           [r_hi[kk], r_lo[kk], col], vals)

        def start_write(h, b):
            pltpu.make_async_copy(
                tbufs.at[b].at[:, :, pl.ds(0, 128)],
                out_hbm.at[h].at[:, w],
                wsems.at[b],
            ).start()

        def wait_write(h, b):
            pltpu.make_async_copy(
                tbufs.at[b].at[:, :, pl.ds(0, 128)],
                out_hbm.at[h].at[:, w],
                wsems.at[b],
            ).wait()

        for b in range(nbuf):
            start_gather(b, b)

        def group(g, carry):
            for b in range(nbuf):
                h = g * nbuf + b
                wait_gather(h, b)

                @pl.when(h >= nbuf)
                def _():
                    wait_write(h - nbuf, b)

                transpose_block(b)
                start_write(h, b)

                @pl.when(h + nbuf < hist)
                def _():
                    start_gather(h + nbuf, b)

            return carry

        lax.fori_loop(0, n_groups, group, 0)
        for b in range(nbuf):
            wait_write(0, b)

    return gather_kernel


def kernel(x, table):
    bsz, hist = x.shape
    vocab, dim = table.shape
    tt = table.T
    xt = x.astype(jnp.int32).T
    n_full = vocab // 128
    tail = vocab - n_full * 128
    if tail:
        t_tailp = jnp.pad(table[n_full * 128:], ((0, 0), (0, 128 - dim)))
        t2 = _build_transpose(vocab, dim)(tt, t_tailp)
    else:
        t2 = _build_transpose(vocab, dim)(tt)
    outp = _build_gather(bsz, hist, vocab, dim)(xt, t2)
    return outp.transpose(2, 4, 0, 1, 3).reshape(bsz, hist, dim)


# speedup vs baseline: 2.1989x; 1.2762x over previous
import functools

import jax
import jax.numpy as jnp
from jax import lax
from jax.experimental import pallas as pl
from jax.experimental.pallas import tpu as pltpu
from jax.experimental.pallas import tpu_sc as plsc

_INFO = plsc.get_sparse_core_info()
_NC = _INFO.num_cores
_NS = _INFO.num_subcores
_NW = _NC * _NS
_L = 16
_PAD = 136


def _mesh():
    return plsc.VectorSubcoreMesh(core_axis_name="c", subcore_axis_name="s")


def _wid():
    return lax.axis_index("s") * _NC + lax.axis_index("c")


def _splat(val):
    return jnp.full((_L,), val, jnp.int32)


@functools.lru_cache(maxsize=None)
def _build_gather(bsz, hist, vocab, dim):
    nbuf = 2
    assert bsz == _NW * 128 and hist % nbuf == 0 and dim % _L == 0
    dr = dim // 8
    n_groups = hist // nbuf

    @functools.partial(
        pl.kernel,
        mesh=_mesh(),
        out_type=jax.ShapeDtypeStruct((hist, dr, bsz // 128, 8, 128),
                                      jnp.float32),
        scratch_types=[
            pltpu.VMEM((hist, 128), jnp.int32),
            pltpu.VMEM((nbuf, 128, 128), jnp.float32),
            pltpu.VMEM((nbuf, dr, 8, _PAD), jnp.float32),
            pltpu.SemaphoreType.DMA((nbuf,)),
            pltpu.SemaphoreType.DMA((nbuf,)),
        ],
        compiler_params=pltpu.CompilerParams(use_tc_tiling_on_sc=True,
                                             needs_layout_passes=False),
    )
    def gather_kernel(xt_hbm, t2_hbm, out_hbm, idx_v, gbufs, tbufs,
                      gsems, wsems):
        w = _wid()
        pltpu.sync_copy(xt_hbm.at[:, pl.ds(w * 128, 128)], idx_v)
        iotas = [lax.iota(jnp.int32, _L) + kk * _L
                 for kk in range(dim // _L)]
        r_hi = [i // 8 for i in iotas]
        r_lo = [i % 8 for i in iotas]

        def start_gather(h, b):
            pltpu.make_async_copy(
                t2_hbm.at[idx_v.at[h]], gbufs.at[b], gsems.at[b]
            ).start()

        def wait_gather(h, b):
            pltpu.make_async_copy(
                t2_hbm.at[idx_v.at[h]], gbufs.at[b], gsems.at[b]
            ).wait()

        def transpose_block(b):
            @plsc.parallel_loop(0, 128, step=4, unroll=4)
            def _(c0):
                for ci in range(4):
                    c = c0 + ci
                    col = _splat(c)
                    for kk in range(dim // _L):
                        vals = gbufs.at[b][c, pl.ds(kk * _L, _L)]
                        plsc.store_scatter(tbufs.at[b],
                                           [r_hi[kk], r_lo[kk], col], vals)

        def start_write(h, b):
            pltpu.make_async_copy(
                tbufs.at[b].at[:, :, pl.ds(0, 128)],
                out_hbm.at[h].at[:, w],
                wsems.at[b],
            ).start()

        def wait_write(h, b):
            pltpu.make_async_copy(
                tbufs.at[b].at[:, :, pl.ds(0, 128)],
                out_hbm.at[h].at[:, w],
                wsems.at[b],
            ).wait()

        for b in range(nbuf):
            start_gather(b, b)

        def group(g, carry):
            for b in range(nbuf):
                h = g * nbuf + b
                wait_gather(h, b)

                @pl.when(h >= nbuf)
                def _():
                    wait_write(h - nbuf, b)

                transpose_block(b)
                start_write(h, b)

                @pl.when(h + nbuf < hist)
                def _():
                    start_gather(h + nbuf, b)

            return carry

        lax.fori_loop(0, n_groups, group, 0)
        for b in range(nbuf):
            wait_write(0, b)

    return gather_kernel


def kernel(x, table):
    bsz, hist = x.shape
    vocab, dim = table.shape
    xt = x.astype(jnp.int32).T
    t2 = jnp.pad(table, ((0, 0), (0, 128 - dim)))
    outp = _build_gather(bsz, hist, vocab, dim)(xt, t2)
    return outp.transpose(2, 4, 0, 1, 3).reshape(bsz, hist, dim)


# speedup vs baseline: 2.2174x; 1.0084x over previous
import functools

import jax
import jax.numpy as jnp
from jax import lax
from jax.experimental import pallas as pl
from jax.experimental.pallas import tpu as pltpu
from jax.experimental.pallas import tpu_sc as plsc

_INFO = plsc.get_sparse_core_info()
_NC = _INFO.num_cores
_NS = _INFO.num_subcores
_NW = _NC * _NS
_NBUF = 4


@functools.lru_cache(maxsize=None)
def _build(bsz, hist, vocab, dim):
    assert bsz % (_NW * _NBUF) == 0
    rows_per_w = bsz // _NW
    n_groups = rows_per_w // _NBUF
    parts = []
    off = 0
    while off < hist:
        ln = min(128, hist - off)
        parts.append((off, ln))
        off += ln
    mesh = plsc.VectorSubcoreMesh(core_axis_name="c", subcore_axis_name="s")

    @functools.partial(
        pl.kernel,
        mesh=mesh,
        out_type=jax.ShapeDtypeStruct((bsz, hist, dim), jnp.float32),
        scratch_types=[
            pltpu.VMEM((rows_per_w, hist), jnp.int32),
            tuple(pltpu.VMEM((_NBUF, ln, dim), jnp.float32) for _, ln in parts),
            pltpu.SemaphoreType.DMA((_NBUF,)),
            pltpu.SemaphoreType.DMA((_NBUF,)),
        ],
        compiler_params=pltpu.CompilerParams(use_tc_tiling_on_sc=False),
    )
    def gather_kernel(x_hbm, table_hbm, out_hbm, idx_v, bufs, gsems, wsems):
        wid = lax.axis_index("s") * _NC + lax.axis_index("c")
        row0 = wid * rows_per_w
        pltpu.sync_copy(x_hbm.at[pl.ds(row0, rows_per_w)], idx_v)

        def start_gather(r, b):
            for p, (off, ln) in enumerate(parts):
                pltpu.make_async_copy(
                    table_hbm.at[idx_v.at[r].at[pl.ds(off, ln)]],
                    bufs[p].at[b],
                    gsems.at[b],
                ).start()

        def wait_gather(r, b):
            for p, (off, ln) in enumerate(parts):
                pltpu.make_async_copy(
                    table_hbm.at[idx_v.at[r].at[pl.ds(off, ln)]],
                    bufs[p].at[b],
                    gsems.at[b],
                ).wait()

        def start_write(r, b):
            for p, (off, ln) in enumerate(parts):
                pltpu.make_async_copy(
                    bufs[p].at[b],
                    out_hbm.at[row0 + r].at[pl.ds(off, ln)],
                    wsems.at[b],
                ).start()

        def wait_write(r, b):
            for p, (off, ln) in enumerate(parts):
                pltpu.make_async_copy(
                    bufs[p].at[b],
                    out_hbm.at[row0 + r].at[pl.ds(off, ln)],
                    wsems.at[b],
                ).wait()

        for b in range(_NBUF):
            start_gather(b, b)

        def group(g, carry):
            base = g * _NBUF
            for b in range(_NBUF):
                wait_gather(base + b, b)
                start_write(base + b, b)
            for b in range(_NBUF):
                wait_write(base + b, b)
                start_gather(base + _NBUF + b, b)
            return carry

        lax.fori_loop(0, n_groups - 1, group, 0)

        base = (n_groups - 1) * _NBUF
        for b in range(_NBUF):
            wait_gather(base + b, b)
            start_write(base + b, b)
        for b in range(_NBUF):
            wait_write(base + b, b)

    return gather_kernel


def kernel(x, table):
    bsz, hist = x.shape
    vocab, dim = table.shape
    return _build(bsz, hist, vocab, dim)(x.astype(jnp.int32), table)
